# baseline (speedup 1.0000x reference)
import jax
import jax.numpy as jnp
from jax.experimental import pallas as pl
from jax.experimental.pallas import tpu as pltpu


def _l2_min_body(ze_hbm, emb_ref, out_ref, zv_ref, sem):
    B = ze_hbm.shape[0]
    copies = [
        pltpu.make_async_copy(ze_hbm.at[b], zv_ref.at[b], sem.at[b])
        for b in range(B)
    ]
    for c in copies:
        c.start()
    e = emb_ref[...]
    en = (e * -2.0).astype(jnp.bfloat16)
    ee = jnp.sum(e * e, axis=1, keepdims=True)
    for b in range(B):
        copies[b].wait()
        z = zv_ref[b]
        dot = jax.lax.dot_general(
            en, z.astype(jnp.bfloat16), (((1,), (0,)), ((), ())),
            preferred_element_type=jnp.float32,
        )
        zz = jnp.sum(z * z, axis=0)
        out_ref[b, :] = jnp.min(dot + ee, axis=0) + zz


def kernel(ze, emb):
    B, Q, N = ze.shape
    K, _ = emb.shape
    return pl.pallas_call(
        _l2_min_body,
        in_specs=[
            pl.BlockSpec(memory_space=pl.ANY),
            pl.BlockSpec((K, Q), lambda: (0, 0)),
        ],
        out_specs=pl.BlockSpec((B, N), lambda: (0, 0)),
        out_shape=jax.ShapeDtypeStruct((B, N), jnp.float32),
        scratch_shapes=[
            pltpu.VMEM((B, Q, N), jnp.float32),
            pltpu.SemaphoreType.DMA((B,)),
        ],
    )(ze, emb)
